# Initial kernel scaffold; baseline (speedup 1.0000x reference)
import jax
import jax.numpy as jnp
from jax.experimental import pallas as pl


def kernel(x_p, x_d, edge_attr_p, edge_attr_d, x_p_batch, edge_index_p, W1a, b1a, W1b, b1b, root1, bias1, W2a, b2a, W2b, b2b, root2, bias2, lin1_w, lin1_b, lin2_w, lin2_b):
    raise NotImplementedError("write your pallas kernel here")


# speedup vs baseline: 1.8815x; 1.8815x over previous
import functools

import jax
import jax.numpy as jnp
from jax import lax
from jax.experimental import pallas as pl
from jax.experimental.pallas import tpu as pltpu
from jax.experimental.pallas import tpu_sc as plsc

_NC = 2
_NS = 16
_NW = _NC * _NS
_C = 128


def _sc_gather(table, idx):
    n_rows, d = table.shape
    e = idx.shape[0]
    per_w = e // _NW
    nchunk = (per_w + _C - 1) // _C
    last_off = per_w - _C
    mesh = plsc.VectorSubcoreMesh(core_axis_name="c", subcore_axis_name="s")

    @functools.partial(
        pl.kernel,
        out_type=jax.ShapeDtypeStruct((e, d), table.dtype),
        mesh=mesh,
        compiler_params=pltpu.CompilerParams(use_tc_tiling_on_sc=False),
        scratch_types=[
            pltpu.VMEM((_C,), jnp.int32),
            pltpu.VMEM((_C, d), table.dtype),
            pltpu.SemaphoreType.DMA,
        ],
    )
    def k(table_hbm, idx_hbm, out_hbm, idx_v, rows_v, sem):
        wid = lax.axis_index("s") * _NC + lax.axis_index("c")
        base = wid * per_w

        def body(j, carry):
            off = base + jnp.minimum(j * _C, last_off)
            pltpu.sync_copy(idx_hbm.at[pl.ds(off, _C)], idx_v)
            pltpu.async_copy(table_hbm.at[idx_v], rows_v, sem).wait()
            pltpu.sync_copy(rows_v, out_hbm.at[pl.ds(off, _C)])
            return carry

        lax.fori_loop(0, nchunk, body, 0, unroll=False)

    return k(table, idx)


def _sc_scatter_add(msg, dst, n_nodes, zeros_hbm):
    e, f = msg.shape
    per_w = e // _NW
    full = per_w // _C
    tail = per_w - full * _C
    rows_per_tile = 640
    mesh = plsc.VectorSubcoreMesh(core_axis_name="c", subcore_axis_name="s")

    @functools.partial(
        pl.kernel,
        out_type=jax.ShapeDtypeStruct((_NC, n_nodes, f), msg.dtype),
        mesh=mesh,
        compiler_params=pltpu.CompilerParams(use_tc_tiling_on_sc=False),
        scratch_types=[
            pltpu.VMEM((_C,), jnp.int32),
            pltpu.VMEM((_C, f), msg.dtype),
            pltpu.VMEM((tail,), jnp.int32) if tail else None,
            pltpu.VMEM((tail, f), msg.dtype) if tail else None,
            pltpu.VMEM_SHARED((n_nodes, f), msg.dtype),
        ],
    )
    def k(msg_hbm, dst_hbm, zero_hbm, out_hbm, idx_v, rows_v, idxt_v, rowst_v,
          acc_sh):
        c = lax.axis_index("c")
        s = lax.axis_index("s")
        wid = s * _NC + c
        base = wid * per_w

        n_full_z = n_nodes // rows_per_tile
        rem_z = n_nodes - n_full_z * rows_per_tile

        @pl.when(s < n_full_z)
        def _():
            pltpu.sync_copy(zero_hbm.at[pl.ds(s * rows_per_tile, rows_per_tile)],
                            acc_sh.at[pl.ds(s * rows_per_tile, rows_per_tile)])

        if rem_z:
            @pl.when(s == n_full_z)
            def _():
                pltpu.sync_copy(zero_hbm.at[pl.ds(n_full_z * rows_per_tile, rem_z)],
                                acc_sh.at[pl.ds(n_full_z * rows_per_tile, rem_z)])

        plsc.subcore_barrier()

        def body(j, carry):
            off = base + j * _C
            pltpu.sync_copy(dst_hbm.at[pl.ds(off, _C)], idx_v)
            pltpu.sync_copy(msg_hbm.at[pl.ds(off, _C)], rows_v)
            pltpu.sync_copy(rows_v, acc_sh.at[idx_v], add=True)
            return carry

        lax.fori_loop(0, full, body, 0, unroll=False)

        if tail:
            off = base + full * _C
            pltpu.sync_copy(dst_hbm.at[pl.ds(off, tail)], idxt_v)
            pltpu.sync_copy(msg_hbm.at[pl.ds(off, tail)], rowst_v)
            pltpu.sync_copy(rowst_v, acc_sh.at[idxt_v], add=True)

        plsc.subcore_barrier()

        @pl.when(s < n_full_z)
        def _():
            pltpu.sync_copy(acc_sh.at[pl.ds(s * rows_per_tile, rows_per_tile)],
                            out_hbm.at[c, pl.ds(s * rows_per_tile, rows_per_tile)])

        if rem_z:
            @pl.when(s == n_full_z)
            def _():
                pltpu.sync_copy(acc_sh.at[pl.ds(n_full_z * rows_per_tile, rem_z)],
                                out_hbm.at[c, pl.ds(n_full_z * rows_per_tile, rem_z)])

    return k(msg, dst, zeros_hbm)


def _msg_body(ea_ref, xj_ref, wa_ref, ba_ref, r_ref, bc_ref, wp_ref, f_ref,
              out_ref):
    h = jnp.dot(ea_ref[...], wa_ref[...], preferred_element_type=jnp.float32)
    h = jnp.maximum(h + ba_ref[...], 0.0)
    hx = jnp.dot(h, r_ref[...], preferred_element_type=jnp.float32) + bc_ref[...]
    t = jnp.dot(xj_ref[...], wp_ref[...], preferred_element_type=jnp.float32)
    out_ref[...] = jnp.dot(hx * t, f_ref[...],
                           preferred_element_type=jnp.float32)


def _tc_msg(ea, xj, wa, ba, wp, out_c, eb):
    e, in_c = xj.shape
    ko = wp.shape[1]
    r = jnp.concatenate(
        [jnp.repeat(jnp.eye(16, dtype=jnp.float32), out_c, axis=1),
         jnp.zeros((16, out_c), jnp.float32)], axis=1)
    bc = jnp.concatenate(
        [jnp.zeros((1, ko - out_c), jnp.float32),
         jnp.ones((1, out_c), jnp.float32)], axis=1)
    fold = jnp.tile(jnp.eye(out_c, dtype=jnp.float32), (17, 1))
    grid = e // eb
    return pl.pallas_call(
        _msg_body,
        grid=(grid,),
        in_specs=[
            pl.BlockSpec((eb, 4), lambda i: (i, 0)),
            pl.BlockSpec((eb, in_c), lambda i: (i, 0)),
            pl.BlockSpec((4, 16), lambda i: (0, 0)),
            pl.BlockSpec((1, 16), lambda i: (0, 0)),
            pl.BlockSpec((16, ko), lambda i: (0, 0)),
            pl.BlockSpec((1, ko), lambda i: (0, 0)),
            pl.BlockSpec((in_c, ko), lambda i: (0, 0)),
            pl.BlockSpec((ko, out_c), lambda i: (0, 0)),
        ],
        out_specs=pl.BlockSpec((eb, out_c), lambda i: (i, 0)),
        out_shape=jax.ShapeDtypeStruct((e, out_c), jnp.float32),
    )(ea, xj, wa, ba.reshape(1, 16), r, bc, wp, fold)


def _node_body(x_ref, a0_ref, a1_ref, root_ref, bias_ref, o_ref):
    v = jnp.dot(x_ref[...], root_ref[...], preferred_element_type=jnp.float32)
    o_ref[...] = jnp.maximum(v + a0_ref[...] + a1_ref[...] + bias_ref[...], 0.0)


def _tc_node(x, a0, a1, root, bias, nb):
    n, in_c = x.shape
    out_c = root.shape[1]
    return pl.pallas_call(
        _node_body,
        grid=(n // nb,),
        in_specs=[
            pl.BlockSpec((nb, in_c), lambda i: (i, 0)),
            pl.BlockSpec((nb, out_c), lambda i: (i, 0)),
            pl.BlockSpec((nb, out_c), lambda i: (i, 0)),
            pl.BlockSpec((in_c, out_c), lambda i: (0, 0)),
            pl.BlockSpec((1, out_c), lambda i: (0, 0)),
        ],
        out_specs=pl.BlockSpec((nb, out_c), lambda i: (i, 0)),
        out_shape=jax.ShapeDtypeStruct((n, out_c), jnp.float32),
    )(x, a0, a1, root, bias.reshape(1, out_c))


def _final_body(x_ref, a0_ref, a1_ref, b_ref, root_ref, bias_ref,
                l1w_ref, l1b_ref, l2w_ref, l2b_ref, o_ref, sums, cnts):
    i = pl.program_id(0)
    nb = x_ref.shape[0]
    g = sums.shape[0]

    @pl.when(i == 0)
    def _():
        sums[...] = jnp.zeros_like(sums)
        cnts[...] = jnp.zeros_like(cnts)

    v = jnp.dot(x_ref[...], root_ref[...], preferred_element_type=jnp.float32)
    x2 = jnp.maximum(v + a0_ref[...] + a1_ref[...] + bias_ref[...], 0.0)
    gio = lax.broadcasted_iota(jnp.int32, (nb, g), 1)
    mask = (gio == b_ref[...]).astype(jnp.float32)
    dn = (((0,), (0,)), ((), ()))
    sums[...] += lax.dot_general(mask, x2, dn,
                                 preferred_element_type=jnp.float32)
    cnts[...] += lax.dot_general(mask, jnp.ones((nb, 1), jnp.float32), dn,
                                 preferred_element_type=jnp.float32)

    @pl.when(i == pl.num_programs(0) - 1)
    def _():
        pooled = sums[...] / jnp.maximum(cnts[...], 1.0)
        o1 = jnp.dot(pooled, l1w_ref[...],
                     preferred_element_type=jnp.float32) + l1b_ref[...]
        o_ref[...] = jnp.dot(o1, l2w_ref[...],
                             preferred_element_type=jnp.float32) + l2b_ref[...]


def _tc_final(x1, a0, a1, batch, root, bias, l1w, l1b, l2w, l2b, g, nb):
    n, in_c = x1.shape
    out_c = root.shape[1]
    return pl.pallas_call(
        _final_body,
        grid=(n // nb,),
        in_specs=[
            pl.BlockSpec((nb, in_c), lambda i: (i, 0)),
            pl.BlockSpec((nb, out_c), lambda i: (i, 0)),
            pl.BlockSpec((nb, out_c), lambda i: (i, 0)),
            pl.BlockSpec((nb, 1), lambda i: (i, 0)),
            pl.BlockSpec((in_c, out_c), lambda i: (0, 0)),
            pl.BlockSpec((1, out_c), lambda i: (0, 0)),
            pl.BlockSpec((16, 8), lambda i: (0, 0)),
            pl.BlockSpec((1, 8), lambda i: (0, 0)),
            pl.BlockSpec((8, 1), lambda i: (0, 0)),
            pl.BlockSpec((1, 1), lambda i: (0, 0)),
        ],
        out_specs=pl.BlockSpec((g, 1), lambda i: (0, 0)),
        out_shape=jax.ShapeDtypeStruct((g, 1), jnp.float32),
        scratch_shapes=[
            pltpu.VMEM((g, out_c), jnp.float32),
            pltpu.VMEM((g, 1), jnp.float32),
        ],
    )(x1, a0, a1, batch, root, bias.reshape(1, out_c),
      l1w, l1b.reshape(1, 8), l2w, l2b.reshape(1, 1))


def _prep_wp(wb, bb, in_c, out_c):
    w2r = wb.reshape(16, in_c, out_c)
    return jnp.concatenate(
        [jnp.transpose(w2r, (1, 0, 2)).reshape(in_c, 16 * out_c),
         bb.reshape(in_c, out_c)], axis=1)


def kernel(x_p, x_d, edge_attr_p, edge_attr_d, x_p_batch, edge_index_p,
           W1a, b1a, W1b, b1b, root1, bias1,
           W2a, b2a, W2b, b2b, root2, bias2,
           lin1_w, lin1_b, lin2_w, lin2_b):
    n, d = x_p.shape
    e = edge_index_p.shape[1]
    g = 64
    src = edge_index_p[0]
    dst = edge_index_p[1]

    wp1 = _prep_wp(W1b, b1b, d, 32)
    wp2 = _prep_wp(W2b, b2b, 32, 16)
    zeros32 = jnp.zeros((n, 32), jnp.float32)
    zeros16 = jnp.zeros((n, 16), jnp.float32)

    xj1 = _sc_gather(x_p, src)
    msg1 = _tc_msg(edge_attr_p, xj1, W1a, b1a, wp1, 32, eb=1000)
    parts1 = _sc_scatter_add(msg1, dst, n, zeros32)
    x1 = _tc_node(x_p, parts1[0], parts1[1], root1, bias1, nb=1000)

    xj2 = _sc_gather(x1, src)
    msg2 = _tc_msg(edge_attr_p, xj2, W2a, b2a, wp2, 16, eb=1000)
    parts2 = _sc_scatter_add(msg2, dst, n, zeros16)

    return _tc_final(x1, parts2[0], parts2[1], x_p_batch.reshape(n, 1),
                     root2, bias2, lin1_w, lin1_b, lin2_w, lin2_b, g, nb=1000)


# speedup vs baseline: 1.8864x; 1.0026x over previous
import functools

import jax
import jax.numpy as jnp
from jax import lax
from jax.experimental import pallas as pl
from jax.experimental.pallas import tpu as pltpu
from jax.experimental.pallas import tpu_sc as plsc

_NC = 2
_NS = 16
_NW = _NC * _NS
_C = 128


def _sc_gather(table, idx):
    n_rows, d = table.shape
    e = idx.shape[0]
    per_w = e // _NW
    nchunk = (per_w + _C - 1) // _C
    last_off = per_w - _C
    mesh = plsc.VectorSubcoreMesh(core_axis_name="c", subcore_axis_name="s")

    @functools.partial(
        pl.kernel,
        out_type=jax.ShapeDtypeStruct((e, d), table.dtype),
        mesh=mesh,
        scratch_types=[
            pltpu.VMEM((_C,), jnp.int32),
            pltpu.VMEM((_C, d), table.dtype),
            pltpu.SemaphoreType.DMA,
        ],
    )
    def k(table_hbm, idx_hbm, out_hbm, idx_v, rows_v, sem):
        wid = lax.axis_index("s") * _NC + lax.axis_index("c")
        base = wid * per_w

        def body(j, carry):
            off = base + jnp.minimum(j * _C, last_off)
            pltpu.sync_copy(idx_hbm.at[pl.ds(off, _C)], idx_v)
            pltpu.async_copy(table_hbm.at[idx_v], rows_v, sem).wait()
            pltpu.sync_copy(rows_v, out_hbm.at[pl.ds(off, _C)])
            return carry

        lax.fori_loop(0, nchunk, body, 0, unroll=False)

    return k(table, idx)


def _sc_scatter_add(msg, dst, n_nodes, zeros_hbm):
    e, f = msg.shape
    per_w = e // _NW
    full = per_w // _C
    tail = per_w - full * _C
    rows_per_tile = 640
    mesh = plsc.VectorSubcoreMesh(core_axis_name="c", subcore_axis_name="s")

    @functools.partial(
        pl.kernel,
        out_type=jax.ShapeDtypeStruct((_NC, n_nodes, f), msg.dtype),
        mesh=mesh,
        compiler_params=pltpu.CompilerParams(use_tc_tiling_on_sc=False),
        scratch_types=[
            pltpu.VMEM((_C,), jnp.int32),
            pltpu.VMEM((_C, f), msg.dtype),
            pltpu.VMEM((tail,), jnp.int32) if tail else None,
            pltpu.VMEM((tail, f), msg.dtype) if tail else None,
            pltpu.VMEM_SHARED((n_nodes, f), msg.dtype),
        ],
    )
    def k(msg_hbm, dst_hbm, zero_hbm, out_hbm, idx_v, rows_v, idxt_v, rowst_v,
          acc_sh):
        c = lax.axis_index("c")
        s = lax.axis_index("s")
        wid = s * _NC + c
        base = wid * per_w

        n_full_z = n_nodes // rows_per_tile
        rem_z = n_nodes - n_full_z * rows_per_tile

        @pl.when(s < n_full_z)
        def _():
            pltpu.sync_copy(zero_hbm.at[pl.ds(s * rows_per_tile, rows_per_tile)],
                            acc_sh.at[pl.ds(s * rows_per_tile, rows_per_tile)])

        if rem_z:
            @pl.when(s == n_full_z)
            def _():
                pltpu.sync_copy(zero_hbm.at[pl.ds(n_full_z * rows_per_tile, rem_z)],
                                acc_sh.at[pl.ds(n_full_z * rows_per_tile, rem_z)])

        plsc.subcore_barrier()

        def body(j, carry):
            off = base + j * _C
            pltpu.sync_copy(dst_hbm.at[pl.ds(off, _C)], idx_v)
            pltpu.sync_copy(msg_hbm.at[pl.ds(off, _C)], rows_v)
            pltpu.sync_copy(rows_v, acc_sh.at[idx_v], add=True)
            return carry

        lax.fori_loop(0, full, body, 0, unroll=False)

        if tail:
            off = base + full * _C
            pltpu.sync_copy(dst_hbm.at[pl.ds(off, tail)], idxt_v)
            pltpu.sync_copy(msg_hbm.at[pl.ds(off, tail)], rowst_v)
            pltpu.sync_copy(rowst_v, acc_sh.at[idxt_v], add=True)

        plsc.subcore_barrier()

        @pl.when(s < n_full_z)
        def _():
            pltpu.sync_copy(acc_sh.at[pl.ds(s * rows_per_tile, rows_per_tile)],
                            out_hbm.at[c, pl.ds(s * rows_per_tile, rows_per_tile)])

        if rem_z:
            @pl.when(s == n_full_z)
            def _():
                pltpu.sync_copy(acc_sh.at[pl.ds(n_full_z * rows_per_tile, rem_z)],
                                out_hbm.at[c, pl.ds(n_full_z * rows_per_tile, rem_z)])

    return k(msg, dst, zeros_hbm)


def _msg_body(in_c, ea_ref, xj_ref, wa_ref, ba_ref, r_ref, bc_ref, wp_ref,
              f_ref, out_ref):
    h = jnp.dot(ea_ref[...].astype(jnp.bfloat16), wa_ref[...],
                preferred_element_type=jnp.float32)
    h = jnp.maximum(h + ba_ref[...], 0.0)
    hx = jnp.dot(h.astype(jnp.bfloat16), r_ref[...],
                 preferred_element_type=jnp.float32) + bc_ref[...]
    xjb = xj_ref[...][:, :in_c].astype(jnp.bfloat16)
    t = jnp.dot(xjb, wp_ref[...], preferred_element_type=jnp.float32)
    out_ref[...] = jnp.dot((hx * t).astype(jnp.bfloat16), f_ref[...],
                           preferred_element_type=jnp.float32)


def _tc_msg(ea, xj, wa, ba, wp, out_c, eb):
    e, w_in = xj.shape
    in_c = wp.shape[0]
    ko = wp.shape[1]
    bf = jnp.bfloat16
    r = jnp.concatenate(
        [jnp.repeat(jnp.eye(16, dtype=jnp.float32), out_c, axis=1),
         jnp.zeros((16, out_c), jnp.float32)], axis=1)
    bc = jnp.concatenate(
        [jnp.zeros((1, ko - out_c), jnp.float32),
         jnp.ones((1, out_c), jnp.float32)], axis=1)
    fold = jnp.tile(jnp.eye(out_c, dtype=jnp.float32), (17, 1))
    grid = e // eb
    return pl.pallas_call(
        functools.partial(_msg_body, in_c),
        grid=(grid,),
        in_specs=[
            pl.BlockSpec((eb, 4), lambda i: (i, 0)),
            pl.BlockSpec((eb, w_in), lambda i: (i, 0)),
            pl.BlockSpec((4, 16), lambda i: (0, 0)),
            pl.BlockSpec((1, 16), lambda i: (0, 0)),
            pl.BlockSpec((16, ko), lambda i: (0, 0)),
            pl.BlockSpec((1, ko), lambda i: (0, 0)),
            pl.BlockSpec((in_c, ko), lambda i: (0, 0)),
            pl.BlockSpec((ko, out_c), lambda i: (0, 0)),
        ],
        out_specs=pl.BlockSpec((eb, out_c), lambda i: (i, 0)),
        out_shape=jax.ShapeDtypeStruct((e, out_c), jnp.float32),
    )(ea, xj, wa.astype(bf), ba.reshape(1, 16), r.astype(bf), bc,
      wp.astype(bf), fold.astype(bf))


def _node_body(x_ref, a0_ref, a1_ref, root_ref, bias_ref, o_ref):
    nb = x_ref.shape[0]
    out_c = root_ref.shape[1]
    v = jnp.dot(x_ref[...], root_ref[...], preferred_element_type=jnp.float32)
    v = jnp.maximum(v + a0_ref[...] + a1_ref[...] + bias_ref[...], 0.0)
    o_ref[...] = jnp.concatenate(
        [v, jnp.zeros((nb, 128 - out_c), jnp.float32)], axis=1)


def _tc_node(x, a0, a1, root, bias, nb):
    n, in_c = x.shape
    out_c = root.shape[1]
    return pl.pallas_call(
        _node_body,
        grid=(n // nb,),
        in_specs=[
            pl.BlockSpec((nb, in_c), lambda i: (i, 0)),
            pl.BlockSpec((nb, out_c), lambda i: (i, 0)),
            pl.BlockSpec((nb, out_c), lambda i: (i, 0)),
            pl.BlockSpec((in_c, out_c), lambda i: (0, 0)),
            pl.BlockSpec((1, out_c), lambda i: (0, 0)),
        ],
        out_specs=pl.BlockSpec((nb, 128), lambda i: (i, 0)),
        out_shape=jax.ShapeDtypeStruct((n, 128), jnp.float32),
    )(x, a0, a1, root, bias.reshape(1, out_c))


def _final_body(x_ref, a0_ref, a1_ref, b_ref, root_ref, bias_ref,
                l1w_ref, l1b_ref, l2w_ref, l2b_ref, o_ref, sums, cnts):
    i = pl.program_id(0)
    nb = x_ref.shape[0]
    g = sums.shape[0]

    @pl.when(i == 0)
    def _():
        sums[...] = jnp.zeros_like(sums)
        cnts[...] = jnp.zeros_like(cnts)

    in_c = root_ref.shape[0]
    v = jnp.dot(x_ref[...][:, :in_c], root_ref[...],
                preferred_element_type=jnp.float32)
    x2 = jnp.maximum(v + a0_ref[...] + a1_ref[...] + bias_ref[...], 0.0)
    gio = lax.broadcasted_iota(jnp.int32, (nb, g), 1)
    mask = (gio == b_ref[...]).astype(jnp.float32)
    dn = (((0,), (0,)), ((), ()))
    sums[...] += lax.dot_general(mask, x2, dn,
                                 preferred_element_type=jnp.float32)
    cnts[...] += lax.dot_general(mask, jnp.ones((nb, 1), jnp.float32), dn,
                                 preferred_element_type=jnp.float32)

    @pl.when(i == pl.num_programs(0) - 1)
    def _():
        pooled = sums[...] / jnp.maximum(cnts[...], 1.0)
        o1 = jnp.dot(pooled, l1w_ref[...],
                     preferred_element_type=jnp.float32) + l1b_ref[...]
        o_ref[...] = jnp.dot(o1, l2w_ref[...],
                             preferred_element_type=jnp.float32) + l2b_ref[...]


def _tc_final(x1, a0, a1, batch, root, bias, l1w, l1b, l2w, l2b, g, nb):
    n, w_in = x1.shape
    in_c, out_c = root.shape
    return pl.pallas_call(
        _final_body,
        grid=(n // nb,),
        in_specs=[
            pl.BlockSpec((nb, w_in), lambda i: (i, 0)),
            pl.BlockSpec((nb, out_c), lambda i: (i, 0)),
            pl.BlockSpec((nb, out_c), lambda i: (i, 0)),
            pl.BlockSpec((nb, 1), lambda i: (i, 0)),
            pl.BlockSpec((in_c, out_c), lambda i: (0, 0)),
            pl.BlockSpec((1, out_c), lambda i: (0, 0)),
            pl.BlockSpec((16, 8), lambda i: (0, 0)),
            pl.BlockSpec((1, 8), lambda i: (0, 0)),
            pl.BlockSpec((8, 1), lambda i: (0, 0)),
            pl.BlockSpec((1, 1), lambda i: (0, 0)),
        ],
        out_specs=pl.BlockSpec((g, 1), lambda i: (0, 0)),
        out_shape=jax.ShapeDtypeStruct((g, 1), jnp.float32),
        scratch_shapes=[
            pltpu.VMEM((g, out_c), jnp.float32),
            pltpu.VMEM((g, 1), jnp.float32),
        ],
    )(x1, a0, a1, batch, root, bias.reshape(1, out_c),
      l1w, l1b.reshape(1, 8), l2w, l2b.reshape(1, 1))


def _prep_wp(wb, bb, in_c, out_c):
    w2r = wb.reshape(16, in_c, out_c)
    return jnp.concatenate(
        [jnp.transpose(w2r, (1, 0, 2)).reshape(in_c, 16 * out_c),
         bb.reshape(in_c, out_c)], axis=1)


def kernel(x_p, x_d, edge_attr_p, edge_attr_d, x_p_batch, edge_index_p,
           W1a, b1a, W1b, b1b, root1, bias1,
           W2a, b2a, W2b, b2b, root2, bias2,
           lin1_w, lin1_b, lin2_w, lin2_b):
    n, d = x_p.shape
    e = edge_index_p.shape[1]
    g = 64
    src = edge_index_p[0]
    dst = edge_index_p[1]

    wp1 = _prep_wp(W1b, b1b, d, 32)
    wp2 = _prep_wp(W2b, b2b, 32, 16)
    zeros32 = jnp.zeros((n, 32), jnp.float32)
    zeros16 = jnp.zeros((n, 16), jnp.float32)

    xj1 = _sc_gather(x_p, src)
    msg1 = _tc_msg(edge_attr_p, xj1, W1a, b1a, wp1, 32, eb=1000)
    parts1 = _sc_scatter_add(msg1, dst, n, zeros32)
    x1 = _tc_node(x_p, parts1[0], parts1[1], root1, bias1, nb=1000)

    xj2 = _sc_gather(x1, src)
    msg2 = _tc_msg(edge_attr_p, xj2, W2a, b2a, wp2, 16, eb=1000)
    parts2 = _sc_scatter_add(msg2, dst, n, zeros16)

    return _tc_final(x1, parts2[0], parts2[1], x_p_batch.reshape(n, 1),
                     root2, bias2, lin1_w, lin1_b, lin2_w, lin2_b, g, nb=1000)


# speedup vs baseline: 2.2765x; 1.2068x over previous
import functools

import jax
import jax.numpy as jnp
from jax import lax
from jax.experimental import pallas as pl
from jax.experimental.pallas import tpu as pltpu
from jax.experimental.pallas import tpu_sc as plsc

_NC = 2
_NS = 16
_NW = _NC * _NS
_C = 128


def _sc_gather(table, idx):
    n_rows, d = table.shape
    e = idx.shape[0]
    per_w = e // _NW
    nchunk = (per_w + _C - 1) // _C
    last_off = per_w - _C
    mesh = plsc.VectorSubcoreMesh(core_axis_name="c", subcore_axis_name="s")

    @functools.partial(
        pl.kernel,
        out_type=jax.ShapeDtypeStruct((e, d), table.dtype),
        mesh=mesh,
        scratch_types=[
            pltpu.VMEM((_C,), jnp.int32),
            pltpu.VMEM((_C, d), table.dtype),
            pltpu.SemaphoreType.DMA,
        ],
    )
    def k(table_hbm, idx_hbm, out_hbm, idx_v, rows_v, sem):
        wid = lax.axis_index("s") * _NC + lax.axis_index("c")
        base = wid * per_w

        def body(j, carry):
            off = base + jnp.minimum(j * _C, last_off)
            pltpu.sync_copy(idx_hbm.at[pl.ds(off, _C)], idx_v)
            pltpu.async_copy(table_hbm.at[idx_v], rows_v, sem).wait()
            pltpu.sync_copy(rows_v, out_hbm.at[pl.ds(off, _C)])
            return carry

        lax.fori_loop(0, nchunk, body, 0, unroll=False)

    return k(table, idx)


def _sc_scatter_add(msg, dst, n_nodes, zeros_hbm):
    e, f = msg.shape
    per_w = e // _NW
    full = per_w // _C
    tail = per_w - full * _C
    rows_per_tile = 640
    mesh = plsc.VectorSubcoreMesh(core_axis_name="c", subcore_axis_name="s")

    @functools.partial(
        pl.kernel,
        out_type=jax.ShapeDtypeStruct((_NC, n_nodes, f), msg.dtype),
        mesh=mesh,
        scratch_types=[
            pltpu.VMEM((_C,), jnp.int32),
            pltpu.VMEM((_C, f), msg.dtype),
            pltpu.VMEM((tail,), jnp.int32) if tail else None,
            pltpu.VMEM((tail, f), msg.dtype) if tail else None,
            pltpu.VMEM_SHARED((n_nodes, f), msg.dtype),
        ],
    )
    def k(msg_hbm, dst_hbm, zero_hbm, out_hbm, idx_v, rows_v, idxt_v, rowst_v,
          acc_sh):
        c = lax.axis_index("c")
        s = lax.axis_index("s")
        wid = s * _NC + c
        base = wid * per_w

        n_full_z = n_nodes // rows_per_tile
        rem_z = n_nodes - n_full_z * rows_per_tile

        @pl.when(s < n_full_z)
        def _():
            pltpu.sync_copy(zero_hbm.at[pl.ds(s * rows_per_tile, rows_per_tile)],
                            acc_sh.at[pl.ds(s * rows_per_tile, rows_per_tile)])

        if rem_z:
            @pl.when(s == n_full_z)
            def _():
                pltpu.sync_copy(zero_hbm.at[pl.ds(n_full_z * rows_per_tile, rem_z)],
                                acc_sh.at[pl.ds(n_full_z * rows_per_tile, rem_z)])

        plsc.subcore_barrier()

        def body(j, carry):
            off = base + j * _C
            pltpu.sync_copy(dst_hbm.at[pl.ds(off, _C)], idx_v)
            pltpu.sync_copy(msg_hbm.at[pl.ds(off, _C)], rows_v)
            pltpu.sync_copy(rows_v, acc_sh.at[idx_v], add=True)
            return carry

        lax.fori_loop(0, full, body, 0, unroll=False)

        if tail:
            off = base + full * _C
            pltpu.sync_copy(dst_hbm.at[pl.ds(off, tail)], idxt_v)
            pltpu.sync_copy(msg_hbm.at[pl.ds(off, tail)], rowst_v)
            pltpu.sync_copy(rowst_v, acc_sh.at[idxt_v], add=True)

        plsc.subcore_barrier()

        @pl.when(s < n_full_z)
        def _():
            pltpu.sync_copy(acc_sh.at[pl.ds(s * rows_per_tile, rows_per_tile)],
                            out_hbm.at[c, pl.ds(s * rows_per_tile, rows_per_tile)])

        if rem_z:
            @pl.when(s == n_full_z)
            def _():
                pltpu.sync_copy(acc_sh.at[pl.ds(n_full_z * rows_per_tile, rem_z)],
                                out_hbm.at[c, pl.ds(n_full_z * rows_per_tile, rem_z)])

    return k(msg, dst, zeros_hbm)


def _msg_body(in_c, ea_ref, xj_ref, wa_ref, ba_ref, r_ref, bc_ref, wp_ref,
              f_ref, out_ref):
    h = jnp.dot(ea_ref[...].astype(jnp.bfloat16), wa_ref[...],
                preferred_element_type=jnp.float32)
    h = jnp.maximum(h + ba_ref[...], 0.0)
    hx = jnp.dot(h.astype(jnp.bfloat16), r_ref[...],
                 preferred_element_type=jnp.float32) + bc_ref[...]
    xjb = xj_ref[...][:, :in_c].astype(jnp.bfloat16)
    t = jnp.dot(xjb, wp_ref[...], preferred_element_type=jnp.float32)
    msg = jnp.dot((hx * t).astype(jnp.bfloat16), f_ref[...],
                  preferred_element_type=jnp.float32)
    eb = msg.shape[0]
    out_ref[...] = jnp.concatenate(
        [msg, jnp.zeros((eb, 128 - msg.shape[1]), jnp.float32)], axis=1)


def _tc_msg(ea, xj, wa, ba, wp, out_c, eb):
    e, w_in = xj.shape
    in_c = wp.shape[0]
    ko = wp.shape[1]
    bf = jnp.bfloat16
    r = jnp.concatenate(
        [jnp.repeat(jnp.eye(16, dtype=jnp.float32), out_c, axis=1),
         jnp.zeros((16, out_c), jnp.float32)], axis=1)
    bc = jnp.concatenate(
        [jnp.zeros((1, ko - out_c), jnp.float32),
         jnp.ones((1, out_c), jnp.float32)], axis=1)
    fold = jnp.tile(jnp.eye(out_c, dtype=jnp.float32), (17, 1))
    grid = e // eb
    return pl.pallas_call(
        functools.partial(_msg_body, in_c),
        grid=(grid,),
        in_specs=[
            pl.BlockSpec((eb, 4), lambda i: (i, 0)),
            pl.BlockSpec((eb, w_in), lambda i: (i, 0)),
            pl.BlockSpec((4, 16), lambda i: (0, 0)),
            pl.BlockSpec((1, 16), lambda i: (0, 0)),
            pl.BlockSpec((16, ko), lambda i: (0, 0)),
            pl.BlockSpec((1, ko), lambda i: (0, 0)),
            pl.BlockSpec((in_c, ko), lambda i: (0, 0)),
            pl.BlockSpec((ko, out_c), lambda i: (0, 0)),
        ],
        out_specs=pl.BlockSpec((eb, 128), lambda i: (i, 0)),
        out_shape=jax.ShapeDtypeStruct((e, 128), jnp.float32),
    )(ea, xj, wa.astype(bf), ba.reshape(1, 16), r.astype(bf), bc,
      wp.astype(bf), fold.astype(bf))


def _node_body(x_ref, a0_ref, a1_ref, root_ref, bias_ref, o_ref):
    nb = x_ref.shape[0]
    out_c = root_ref.shape[1]
    v = jnp.dot(x_ref[...], root_ref[...], preferred_element_type=jnp.float32)
    v = jnp.maximum(v + a0_ref[...][:, :out_c] + a1_ref[...][:, :out_c]
                    + bias_ref[...], 0.0)
    o_ref[...] = jnp.concatenate(
        [v, jnp.zeros((nb, 128 - out_c), jnp.float32)], axis=1)


def _tc_node(x, a0, a1, root, bias, nb):
    n, in_c = x.shape
    out_c = root.shape[1]
    return pl.pallas_call(
        _node_body,
        grid=(n // nb,),
        in_specs=[
            pl.BlockSpec((nb, in_c), lambda i: (i, 0)),
            pl.BlockSpec((nb, 128), lambda i: (i, 0)),
            pl.BlockSpec((nb, 128), lambda i: (i, 0)),
            pl.BlockSpec((in_c, out_c), lambda i: (0, 0)),
            pl.BlockSpec((1, out_c), lambda i: (0, 0)),
        ],
        out_specs=pl.BlockSpec((nb, 128), lambda i: (i, 0)),
        out_shape=jax.ShapeDtypeStruct((n, 128), jnp.float32),
    )(x, a0, a1, root, bias.reshape(1, out_c))


def _final_body(x_ref, a0_ref, a1_ref, b_ref, root_ref, bias_ref,
                l1w_ref, l1b_ref, l2w_ref, l2b_ref, o_ref, sums, cnts):
    i = pl.program_id(0)
    nb = x_ref.shape[0]
    g = sums.shape[0]

    @pl.when(i == 0)
    def _():
        sums[...] = jnp.zeros_like(sums)
        cnts[...] = jnp.zeros_like(cnts)

    in_c = root_ref.shape[0]
    out_c = root_ref.shape[1]
    v = jnp.dot(x_ref[...][:, :in_c], root_ref[...],
                preferred_element_type=jnp.float32)
    x2 = jnp.maximum(v + a0_ref[...][:, :out_c] + a1_ref[...][:, :out_c]
                     + bias_ref[...], 0.0)
    gio = lax.broadcasted_iota(jnp.int32, (nb, g), 1)
    mask = (gio == b_ref[...]).astype(jnp.float32)
    dn = (((0,), (0,)), ((), ()))
    sums[...] += lax.dot_general(mask, x2, dn,
                                 preferred_element_type=jnp.float32)
    cnts[...] += lax.dot_general(mask, jnp.ones((nb, 1), jnp.float32), dn,
                                 preferred_element_type=jnp.float32)

    @pl.when(i == pl.num_programs(0) - 1)
    def _():
        pooled = sums[...] / jnp.maximum(cnts[...], 1.0)
        o1 = jnp.dot(pooled, l1w_ref[...],
                     preferred_element_type=jnp.float32) + l1b_ref[...]
        o_ref[...] = jnp.dot(o1, l2w_ref[...],
                             preferred_element_type=jnp.float32) + l2b_ref[...]


def _tc_final(x1, a0, a1, batch, root, bias, l1w, l1b, l2w, l2b, g, nb):
    n, w_in = x1.shape
    in_c, out_c = root.shape
    return pl.pallas_call(
        _final_body,
        grid=(n // nb,),
        in_specs=[
            pl.BlockSpec((nb, w_in), lambda i: (i, 0)),
            pl.BlockSpec((nb, 128), lambda i: (i, 0)),
            pl.BlockSpec((nb, 128), lambda i: (i, 0)),
            pl.BlockSpec((nb, 1), lambda i: (i, 0)),
            pl.BlockSpec((in_c, out_c), lambda i: (0, 0)),
            pl.BlockSpec((1, out_c), lambda i: (0, 0)),
            pl.BlockSpec((16, 8), lambda i: (0, 0)),
            pl.BlockSpec((1, 8), lambda i: (0, 0)),
            pl.BlockSpec((8, 1), lambda i: (0, 0)),
            pl.BlockSpec((1, 1), lambda i: (0, 0)),
        ],
        out_specs=pl.BlockSpec((g, 1), lambda i: (0, 0)),
        out_shape=jax.ShapeDtypeStruct((g, 1), jnp.float32),
        scratch_shapes=[
            pltpu.VMEM((g, out_c), jnp.float32),
            pltpu.VMEM((g, 1), jnp.float32),
        ],
    )(x1, a0, a1, batch, root, bias.reshape(1, out_c),
      l1w, l1b.reshape(1, 8), l2w, l2b.reshape(1, 1))


def _prep_wp(wb, bb, in_c, out_c):
    w2r = wb.reshape(16, in_c, out_c)
    return jnp.concatenate(
        [jnp.transpose(w2r, (1, 0, 2)).reshape(in_c, 16 * out_c),
         bb.reshape(in_c, out_c)], axis=1)


def kernel(x_p, x_d, edge_attr_p, edge_attr_d, x_p_batch, edge_index_p,
           W1a, b1a, W1b, b1b, root1, bias1,
           W2a, b2a, W2b, b2b, root2, bias2,
           lin1_w, lin1_b, lin2_w, lin2_b):
    n, d = x_p.shape
    e = edge_index_p.shape[1]
    g = 64
    src = edge_index_p[0]
    dst = edge_index_p[1]

    wp1 = _prep_wp(W1b, b1b, d, 32)
    wp2 = _prep_wp(W2b, b2b, 32, 16)
    zeros128 = jnp.zeros((n, 128), jnp.float32)

    xj1 = _sc_gather(x_p, src)
    msg1 = _tc_msg(edge_attr_p, xj1, W1a, b1a, wp1, 32, eb=1600)
    parts1 = _sc_scatter_add(msg1, dst, n, zeros128)
    x1 = _tc_node(x_p, parts1[0], parts1[1], root1, bias1, nb=1000)

    xj2 = _sc_gather(x1, src)
    msg2 = _tc_msg(edge_attr_p, xj2, W2a, b2a, wp2, 16, eb=1600)
    parts2 = _sc_scatter_add(msg2, dst, n, zeros128)

    return _tc_final(x1, parts2[0], parts2[1], x_p_batch.reshape(n, 1),
                     root2, bias2, lin1_w, lin1_b, lin2_w, lin2_b, g, nb=1000)


# speedup vs baseline: 2.3459x; 1.0305x over previous
import functools

import jax
import jax.numpy as jnp
from jax import lax
from jax.experimental import pallas as pl
from jax.experimental.pallas import tpu as pltpu
from jax.experimental.pallas import tpu_sc as plsc

_NC = 2
_NS = 16
_NW = _NC * _NS
_C = 128


def _sc_gather(table, idx):
    n_rows, d = table.shape
    e = idx.shape[0]
    per_w = e // _NW
    nchunk = (per_w + _C - 1) // _C
    last_off = per_w - _C
    mesh = plsc.VectorSubcoreMesh(core_axis_name="c", subcore_axis_name="s")

    @functools.partial(
        pl.kernel,
        out_type=jax.ShapeDtypeStruct((e, d), table.dtype),
        mesh=mesh,
        scratch_types=[
            pltpu.VMEM((_C,), jnp.int32),
            pltpu.VMEM((_C, d), table.dtype),
            pltpu.SemaphoreType.DMA,
        ],
    )
    def k(table_hbm, idx_hbm, out_hbm, idx_v, rows_v, sem):
        wid = lax.axis_index("s") * _NC + lax.axis_index("c")
        base = wid * per_w

        def body(j, carry):
            off = base + jnp.minimum(j * _C, last_off)
            pltpu.sync_copy(idx_hbm.at[pl.ds(off, _C)], idx_v)
            pltpu.async_copy(table_hbm.at[idx_v], rows_v, sem).wait()
            pltpu.sync_copy(rows_v, out_hbm.at[pl.ds(off, _C)])
            return carry

        lax.fori_loop(0, nchunk, body, 0, unroll=False)

    return k(table, idx)


def _sc_scatter_add(msg, dst, n_nodes, zeros_hbm):
    e, f = msg.shape
    per_w = e // _NW
    full = per_w // _C
    tail = per_w - full * _C
    rows_per_tile = 640
    mesh = plsc.VectorSubcoreMesh(core_axis_name="c", subcore_axis_name="s")

    @functools.partial(
        pl.kernel,
        out_type=jax.ShapeDtypeStruct((_NC, n_nodes, f), msg.dtype),
        mesh=mesh,
        scratch_types=[
            pltpu.VMEM((_C,), jnp.int32),
            pltpu.VMEM((_C, f), msg.dtype),
            pltpu.VMEM((tail,), jnp.int32) if tail else None,
            pltpu.VMEM((tail, f), msg.dtype) if tail else None,
            pltpu.VMEM_SHARED((n_nodes, f), msg.dtype),
        ],
    )
    def k(msg_hbm, dst_hbm, zero_hbm, out_hbm, idx_v, rows_v, idxt_v, rowst_v,
          acc_sh):
        c = lax.axis_index("c")
        s = lax.axis_index("s")
        wid = s * _NC + c
        base = wid * per_w

        n_full_z = n_nodes // rows_per_tile
        rem_z = n_nodes - n_full_z * rows_per_tile

        @pl.when(s < n_full_z)
        def _():
            pltpu.sync_copy(zero_hbm.at[pl.ds(s * rows_per_tile, rows_per_tile)],
                            acc_sh.at[pl.ds(s * rows_per_tile, rows_per_tile)])

        if rem_z:
            @pl.when(s == n_full_z)
            def _():
                pltpu.sync_copy(zero_hbm.at[pl.ds(n_full_z * rows_per_tile, rem_z)],
                                acc_sh.at[pl.ds(n_full_z * rows_per_tile, rem_z)])

        plsc.subcore_barrier()

        def body(j, carry):
            off = base + j * _C
            pltpu.sync_copy(dst_hbm.at[pl.ds(off, _C)], idx_v)
            pltpu.sync_copy(msg_hbm.at[pl.ds(off, _C)], rows_v)
            pltpu.sync_copy(rows_v, acc_sh.at[idx_v], add=True)
            return carry

        lax.fori_loop(0, full, body, 0, unroll=False)

        if tail:
            off = base + full * _C
            pltpu.sync_copy(dst_hbm.at[pl.ds(off, tail)], idxt_v)
            pltpu.sync_copy(msg_hbm.at[pl.ds(off, tail)], rowst_v)
            pltpu.sync_copy(rowst_v, acc_sh.at[idxt_v], add=True)

        plsc.subcore_barrier()

        @pl.when(s < n_full_z)
        def _():
            pltpu.sync_copy(acc_sh.at[pl.ds(s * rows_per_tile, rows_per_tile)],
                            out_hbm.at[c, pl.ds(s * rows_per_tile, rows_per_tile)])

        if rem_z:
            @pl.when(s == n_full_z)
            def _():
                pltpu.sync_copy(acc_sh.at[pl.ds(n_full_z * rows_per_tile, rem_z)],
                                out_hbm.at[c, pl.ds(n_full_z * rows_per_tile, rem_z)])

    return k(msg, dst, zeros_hbm)


def _msg_body(in_c, eat_ref, xj_ref, wrep_ref, brep_ref, wp_ref, f_ref,
              out_ref):
    dn = (((0,), (0,)), ((), ()))
    hx = jnp.maximum(
        lax.dot_general(eat_ref[...].astype(jnp.bfloat16), wrep_ref[...], dn,
                        preferred_element_type=jnp.float32) + brep_ref[...],
        0.0)
    xjb = xj_ref[...][:, :in_c].astype(jnp.bfloat16)
    t = jnp.dot(xjb, wp_ref[...], preferred_element_type=jnp.float32)
    msg = jnp.dot((hx * t).astype(jnp.bfloat16), f_ref[...],
                  preferred_element_type=jnp.float32)
    eb = msg.shape[0]
    out_ref[...] = jnp.concatenate(
        [msg, jnp.zeros((eb, 128 - msg.shape[1]), jnp.float32)], axis=1)


def _tc_msg(ea_t, xj, wa, ba, wp, out_c, eb):
    e, w_in = xj.shape
    in_c = wp.shape[0]
    ko = wp.shape[1]
    bf = jnp.bfloat16
    wrep = jnp.concatenate(
        [jnp.repeat(wa, out_c, axis=1), jnp.zeros((4, out_c), jnp.float32)],
        axis=1)
    brep = jnp.concatenate(
        [jnp.repeat(ba.reshape(1, 16), out_c, axis=1),
         jnp.ones((1, out_c), jnp.float32)], axis=1)
    fold = jnp.tile(jnp.eye(out_c, dtype=jnp.float32), (17, 1))
    grid = e // eb
    return pl.pallas_call(
        functools.partial(_msg_body, in_c),
        grid=(grid,),
        in_specs=[
            pl.BlockSpec((4, eb), lambda i: (0, i)),
            pl.BlockSpec((eb, w_in), lambda i: (i, 0)),
            pl.BlockSpec((4, ko), lambda i: (0, 0)),
            pl.BlockSpec((1, ko), lambda i: (0, 0)),
            pl.BlockSpec((in_c, ko), lambda i: (0, 0)),
            pl.BlockSpec((ko, out_c), lambda i: (0, 0)),
        ],
        out_specs=pl.BlockSpec((eb, 128), lambda i: (i, 0)),
        out_shape=jax.ShapeDtypeStruct((e, 128), jnp.float32),
    )(ea_t, xj, wrep.astype(bf), brep, wp.astype(bf), fold.astype(bf))


def _node_body(x_ref, a0_ref, a1_ref, root_ref, bias_ref, o_ref):
    nb = x_ref.shape[0]
    out_c = root_ref.shape[1]
    v = jnp.dot(x_ref[...], root_ref[...], preferred_element_type=jnp.float32)
    v = jnp.maximum(v + a0_ref[...][:, :out_c] + a1_ref[...][:, :out_c]
                    + bias_ref[...], 0.0)
    o_ref[...] = jnp.concatenate(
        [v, jnp.zeros((nb, 128 - out_c), jnp.float32)], axis=1)


def _tc_node(x, a0, a1, root, bias, nb):
    n, in_c = x.shape
    out_c = root.shape[1]
    return pl.pallas_call(
        _node_body,
        grid=(n // nb,),
        in_specs=[
            pl.BlockSpec((nb, in_c), lambda i: (i, 0)),
            pl.BlockSpec((nb, 128), lambda i: (i, 0)),
            pl.BlockSpec((nb, 128), lambda i: (i, 0)),
            pl.BlockSpec((in_c, out_c), lambda i: (0, 0)),
            pl.BlockSpec((1, out_c), lambda i: (0, 0)),
        ],
        out_specs=pl.BlockSpec((nb, 128), lambda i: (i, 0)),
        out_shape=jax.ShapeDtypeStruct((n, 128), jnp.float32),
    )(x, a0, a1, root, bias.reshape(1, out_c))


def _final_body(x_ref, a0_ref, a1_ref, b_ref, root_ref, bias_ref,
                l1w_ref, l1b_ref, l2w_ref, l2b_ref, o_ref, sums, cnts):
    i = pl.program_id(0)
    nb = x_ref.shape[0]
    g = sums.shape[0]

    @pl.when(i == 0)
    def _():
        sums[...] = jnp.zeros_like(sums)
        cnts[...] = jnp.zeros_like(cnts)

    in_c = root_ref.shape[0]
    out_c = root_ref.shape[1]
    v = jnp.dot(x_ref[...][:, :in_c], root_ref[...],
                preferred_element_type=jnp.float32)
    x2 = jnp.maximum(v + a0_ref[...][:, :out_c] + a1_ref[...][:, :out_c]
                     + bias_ref[...], 0.0)
    gio = lax.broadcasted_iota(jnp.int32, (nb, g), 1)
    mask = (gio == b_ref[...]).astype(jnp.float32)
    dn = (((0,), (0,)), ((), ()))
    sums[...] += lax.dot_general(mask, x2, dn,
                                 preferred_element_type=jnp.float32)
    cnts[...] += lax.dot_general(mask, jnp.ones((nb, 1), jnp.float32), dn,
                                 preferred_element_type=jnp.float32)

    @pl.when(i == pl.num_programs(0) - 1)
    def _():
        pooled = sums[...] / jnp.maximum(cnts[...], 1.0)
        o1 = jnp.dot(pooled, l1w_ref[...],
                     preferred_element_type=jnp.float32) + l1b_ref[...]
        o_ref[...] = jnp.dot(o1, l2w_ref[...],
                             preferred_element_type=jnp.float32) + l2b_ref[...]


def _tc_final(x1, a0, a1, batch, root, bias, l1w, l1b, l2w, l2b, g, nb):
    n, w_in = x1.shape
    in_c, out_c = root.shape
    return pl.pallas_call(
        _final_body,
        grid=(n // nb,),
        in_specs=[
            pl.BlockSpec((nb, w_in), lambda i: (i, 0)),
            pl.BlockSpec((nb, 128), lambda i: (i, 0)),
            pl.BlockSpec((nb, 128), lambda i: (i, 0)),
            pl.BlockSpec((nb, 1), lambda i: (i, 0)),
            pl.BlockSpec((in_c, out_c), lambda i: (0, 0)),
            pl.BlockSpec((1, out_c), lambda i: (0, 0)),
            pl.BlockSpec((16, 8), lambda i: (0, 0)),
            pl.BlockSpec((1, 8), lambda i: (0, 0)),
            pl.BlockSpec((8, 1), lambda i: (0, 0)),
            pl.BlockSpec((1, 1), lambda i: (0, 0)),
        ],
        out_specs=pl.BlockSpec((g, 1), lambda i: (0, 0)),
        out_shape=jax.ShapeDtypeStruct((g, 1), jnp.float32),
        scratch_shapes=[
            pltpu.VMEM((g, out_c), jnp.float32),
            pltpu.VMEM((g, 1), jnp.float32),
        ],
    )(x1, a0, a1, batch, root, bias.reshape(1, out_c),
      l1w, l1b.reshape(1, 8), l2w, l2b.reshape(1, 1))


def _prep_wp(wb, bb, in_c, out_c):
    w2r = wb.reshape(16, in_c, out_c)
    return jnp.concatenate(
        [jnp.transpose(w2r, (1, 0, 2)).reshape(in_c, 16 * out_c),
         bb.reshape(in_c, out_c)], axis=1)


def kernel(x_p, x_d, edge_attr_p, edge_attr_d, x_p_batch, edge_index_p,
           W1a, b1a, W1b, b1b, root1, bias1,
           W2a, b2a, W2b, b2b, root2, bias2,
           lin1_w, lin1_b, lin2_w, lin2_b):
    n, d = x_p.shape
    e = edge_index_p.shape[1]
    g = 64
    src = edge_index_p[0]
    dst = edge_index_p[1]

    wp1 = _prep_wp(W1b, b1b, d, 32)
    wp2 = _prep_wp(W2b, b2b, 32, 16)
    zeros128 = jnp.zeros((n, 128), jnp.float32)

    ea_t = edge_attr_p.T

    xj1 = _sc_gather(x_p, src)
    msg1 = _tc_msg(ea_t, xj1, W1a, b1a, wp1, 32, eb=1280)
    parts1 = _sc_scatter_add(msg1, dst, n, zeros128)
    x1 = _tc_node(x_p, parts1[0], parts1[1], root1, bias1, nb=1000)

    xj2 = _sc_gather(x1, src)
    msg2 = _tc_msg(ea_t, xj2, W2a, b2a, wp2, 16, eb=1280)
    parts2 = _sc_scatter_add(msg2, dst, n, zeros128)

    return _tc_final(x1, parts2[0], parts2[1], x_p_batch.reshape(n, 1),
                     root2, bias2, lin1_w, lin1_b, lin2_w, lin2_b, g, nb=1000)


# speedup vs baseline: 2.9063x; 1.2389x over previous
import functools

import jax
import jax.numpy as jnp
from jax import lax
from jax.experimental import pallas as pl
from jax.experimental.pallas import tpu as pltpu
from jax.experimental.pallas import tpu_sc as plsc

_NC = 2
_NS = 16
_NW = _NC * _NS
_C = 128


def _sc_gather(table, idx):
    n_rows, d = table.shape
    e = idx.shape[0]
    per_w = e // _NW
    nchunk = (per_w + _C - 1) // _C
    last_off = per_w - _C
    mesh = plsc.VectorSubcoreMesh(core_axis_name="c", subcore_axis_name="s")

    @functools.partial(
        pl.kernel,
        out_type=jax.ShapeDtypeStruct((e, d), table.dtype),
        mesh=mesh,
        scratch_types=[
            pltpu.VMEM((_C,), jnp.int32),
            pltpu.VMEM((_C, d), table.dtype),
            pltpu.SemaphoreType.DMA,
        ],
    )
    def k(table_hbm, idx_hbm, out_hbm, idx_v, rows_v, sem):
        wid = lax.axis_index("s") * _NC + lax.axis_index("c")
        base = wid * per_w

        def body(j, carry):
            off = base + jnp.minimum(j * _C, last_off)
            pltpu.sync_copy(idx_hbm.at[pl.ds(off, _C)], idx_v)
            pltpu.async_copy(table_hbm.at[idx_v], rows_v, sem).wait()
            pltpu.sync_copy(rows_v, out_hbm.at[pl.ds(off, _C)])
            return carry

        lax.fori_loop(0, nchunk, body, 0, unroll=False)

    return k(table, idx)


def _sc_scatter_add(msg, dst, n_nodes, zeros_hbm):
    e, f = msg.shape
    per_w = e // _NW
    full = per_w // _C
    tail = per_w - full * _C
    rows_per_tile = 640
    mesh = plsc.VectorSubcoreMesh(core_axis_name="c", subcore_axis_name="s")

    scratch = [
        pltpu.VMEM((_C,), jnp.int32),
        pltpu.VMEM((_C, f), msg.dtype),
    ]
    if tail:
        scratch += [pltpu.VMEM((tail,), jnp.int32),
                    pltpu.VMEM((tail, f), msg.dtype)]
    scratch.append(pltpu.VMEM_SHARED((n_nodes, f), msg.dtype))

    @functools.partial(
        pl.kernel,
        out_type=jax.ShapeDtypeStruct((_NC, n_nodes, f), msg.dtype),
        mesh=mesh,
        scratch_types=scratch,
    )
    def k(msg_hbm, dst_hbm, zero_hbm, out_hbm, idx_v, rows_v, *rest):
        if tail:
            idxt_v, rowst_v, acc_sh = rest
        else:
            acc_sh, = rest
        c = lax.axis_index("c")
        s = lax.axis_index("s")
        wid = s * _NC + c
        base = wid * per_w

        n_full_z = n_nodes // rows_per_tile
        rem_z = n_nodes - n_full_z * rows_per_tile

        @pl.when(s < n_full_z)
        def _():
            pltpu.sync_copy(zero_hbm.at[pl.ds(s * rows_per_tile, rows_per_tile)],
                            acc_sh.at[pl.ds(s * rows_per_tile, rows_per_tile)])

        if rem_z:
            @pl.when(s == n_full_z)
            def _():
                pltpu.sync_copy(zero_hbm.at[pl.ds(n_full_z * rows_per_tile, rem_z)],
                                acc_sh.at[pl.ds(n_full_z * rows_per_tile, rem_z)])

        plsc.subcore_barrier()

        def body(j, carry):
            off = base + j * _C
            pltpu.sync_copy(dst_hbm.at[pl.ds(off, _C)], idx_v)
            pltpu.sync_copy(msg_hbm.at[pl.ds(off, _C)], rows_v)
            pltpu.sync_copy(rows_v, acc_sh.at[idx_v], add=True)
            return carry

        lax.fori_loop(0, full, body, 0, unroll=False)

        if tail:
            off = base + full * _C
            pltpu.sync_copy(dst_hbm.at[pl.ds(off, tail)], idxt_v)
            pltpu.sync_copy(msg_hbm.at[pl.ds(off, tail)], rowst_v)
            pltpu.sync_copy(rowst_v, acc_sh.at[idxt_v], add=True)

        plsc.subcore_barrier()

        @pl.when(s < n_full_z)
        def _():
            pltpu.sync_copy(acc_sh.at[pl.ds(s * rows_per_tile, rows_per_tile)],
                            out_hbm.at[c, pl.ds(s * rows_per_tile, rows_per_tile)])

        if rem_z:
            @pl.when(s == n_full_z)
            def _():
                pltpu.sync_copy(acc_sh.at[pl.ds(n_full_z * rows_per_tile, rem_z)],
                                out_hbm.at[c, pl.ds(n_full_z * rows_per_tile, rem_z)])

    return k(msg, dst, zeros_hbm)


def _msg_body(in_c, eat_ref, xj_ref, wrep_ref, brep_ref, wp_ref, f_ref,
              out_ref):
    dn = (((0,), (0,)), ((), ()))
    hx = jnp.maximum(
        lax.dot_general(eat_ref[...].astype(jnp.bfloat16), wrep_ref[...], dn,
                        preferred_element_type=jnp.float32) + brep_ref[...],
        0.0)
    xjb = xj_ref[...][:, :in_c].astype(jnp.bfloat16)
    t = jnp.dot(xjb, wp_ref[...], preferred_element_type=jnp.float32)
    msg = jnp.dot((hx * t).astype(jnp.bfloat16), f_ref[...],
                  preferred_element_type=jnp.float32)
    eb = msg.shape[0]
    out_ref[...] = jnp.concatenate(
        [msg, jnp.zeros((eb, 128 - msg.shape[1]), jnp.float32)], axis=1)


def _tc_msg(ea_t, xj, wa, ba, wp, out_c, eb):
    e, w_in = xj.shape
    in_c = wp.shape[0]
    ko = wp.shape[1]
    bf = jnp.bfloat16
    wrep = jnp.concatenate(
        [jnp.repeat(wa, out_c, axis=1), jnp.zeros((4, out_c), jnp.float32)],
        axis=1)
    brep = jnp.concatenate(
        [jnp.repeat(ba.reshape(1, 16), out_c, axis=1),
         jnp.ones((1, out_c), jnp.float32)], axis=1)
    fold = jnp.tile(jnp.eye(out_c, dtype=jnp.float32), (17, 1))
    grid = e // eb
    return pl.pallas_call(
        functools.partial(_msg_body, in_c),
        grid=(grid,),
        in_specs=[
            pl.BlockSpec((4, eb), lambda i: (0, i)),
            pl.BlockSpec((eb, w_in), lambda i: (i, 0)),
            pl.BlockSpec((4, ko), lambda i: (0, 0)),
            pl.BlockSpec((1, ko), lambda i: (0, 0)),
            pl.BlockSpec((in_c, ko), lambda i: (0, 0)),
            pl.BlockSpec((ko, out_c), lambda i: (0, 0)),
        ],
        out_specs=pl.BlockSpec((eb, 128), lambda i: (i, 0)),
        out_shape=jax.ShapeDtypeStruct((e, 128), jnp.float32),
    )(ea_t, xj, wrep.astype(bf), brep, wp.astype(bf), fold.astype(bf))


def _node_body(n_agg, x_ref, *refs):
    aggs = refs[:n_agg]
    root_ref, bias_ref, o_ref = refs[n_agg:]
    nb = x_ref.shape[0]
    out_c = root_ref.shape[1]
    v = jnp.dot(x_ref[...], root_ref[...], preferred_element_type=jnp.float32)
    for a in aggs:
        v = v + a[...][:, :out_c]
    v = jnp.maximum(v + bias_ref[...], 0.0)
    o_ref[...] = jnp.concatenate(
        [v, jnp.zeros((nb, 128 - out_c), jnp.float32)], axis=1)


def _tc_node(x, aggs, root, bias, nb):
    n, in_c = x.shape
    out_c = root.shape[1]
    return pl.pallas_call(
        functools.partial(_node_body, len(aggs)),
        grid=(n // nb,),
        in_specs=[pl.BlockSpec((nb, in_c), lambda i: (i, 0))]
        + [pl.BlockSpec((nb, 128), lambda i: (i, 0)) for _ in aggs]
        + [
            pl.BlockSpec((in_c, out_c), lambda i: (0, 0)),
            pl.BlockSpec((1, out_c), lambda i: (0, 0)),
        ],
        out_specs=pl.BlockSpec((nb, 128), lambda i: (i, 0)),
        out_shape=jax.ShapeDtypeStruct((n, 128), jnp.float32),
    )(x, *aggs, root, bias.reshape(1, out_c))


def _final_body(n_agg, x_ref, *refs):
    aggs = refs[:n_agg]
    (b_ref, root_ref, bias_ref, l1w_ref, l1b_ref, l2w_ref, l2b_ref,
     o_ref, sums, cnts) = refs[n_agg:]
    i = pl.program_id(0)
    nb = x_ref.shape[0]
    g = sums.shape[0]

    @pl.when(i == 0)
    def _():
        sums[...] = jnp.zeros_like(sums)
        cnts[...] = jnp.zeros_like(cnts)

    in_c = root_ref.shape[0]
    out_c = root_ref.shape[1]
    v = jnp.dot(x_ref[...][:, :in_c], root_ref[...],
                preferred_element_type=jnp.float32)
    for a in aggs:
        v = v + a[...][:, :out_c]
    x2 = jnp.maximum(v + bias_ref[...], 0.0)
    gio = lax.broadcasted_iota(jnp.int32, (nb, g), 1)
    mask = (gio == b_ref[...]).astype(jnp.float32)
    dn = (((0,), (0,)), ((), ()))
    sums[...] += lax.dot_general(mask, x2, dn,
                                 preferred_element_type=jnp.float32)
    cnts[...] += lax.dot_general(mask, jnp.ones((nb, 1), jnp.float32), dn,
                                 preferred_element_type=jnp.float32)

    @pl.when(i == pl.num_programs(0) - 1)
    def _():
        pooled = sums[...] / jnp.maximum(cnts[...], 1.0)
        o1 = jnp.dot(pooled, l1w_ref[...],
                     preferred_element_type=jnp.float32) + l1b_ref[...]
        o_ref[...] = jnp.dot(o1, l2w_ref[...],
                             preferred_element_type=jnp.float32) + l2b_ref[...]


def _tc_final(x1, aggs, batch, root, bias, l1w, l1b, l2w, l2b, g, nb):
    n, w_in = x1.shape
    in_c, out_c = root.shape
    return pl.pallas_call(
        functools.partial(_final_body, len(aggs)),
        grid=(n // nb,),
        in_specs=[pl.BlockSpec((nb, w_in), lambda i: (i, 0))]
        + [pl.BlockSpec((nb, 128), lambda i: (i, 0)) for _ in aggs]
        + [
            pl.BlockSpec((nb, 1), lambda i: (i, 0)),
            pl.BlockSpec((in_c, out_c), lambda i: (0, 0)),
            pl.BlockSpec((1, out_c), lambda i: (0, 0)),
            pl.BlockSpec((16, 8), lambda i: (0, 0)),
            pl.BlockSpec((1, 8), lambda i: (0, 0)),
            pl.BlockSpec((8, 1), lambda i: (0, 0)),
            pl.BlockSpec((1, 1), lambda i: (0, 0)),
        ],
        out_specs=pl.BlockSpec((g, 1), lambda i: (0, 0)),
        out_shape=jax.ShapeDtypeStruct((g, 1), jnp.float32),
        scratch_shapes=[
            pltpu.VMEM((g, out_c), jnp.float32),
            pltpu.VMEM((g, 1), jnp.float32),
        ],
    )(x1, *aggs, batch, root, bias.reshape(1, out_c),
      l1w, l1b.reshape(1, 8), l2w, l2b.reshape(1, 1))


def _prep_wp(wb, bb, in_c, out_c):
    w2r = wb.reshape(16, in_c, out_c)
    return jnp.concatenate(
        [jnp.transpose(w2r, (1, 0, 2)).reshape(in_c, 16 * out_c),
         bb.reshape(in_c, out_c)], axis=1)


def kernel(x_p, x_d, edge_attr_p, edge_attr_d, x_p_batch, edge_index_p,
           W1a, b1a, W1b, b1b, root1, bias1,
           W2a, b2a, W2b, b2b, root2, bias2,
           lin1_w, lin1_b, lin2_w, lin2_b):
    n, d = x_p.shape
    e = edge_index_p.shape[1]
    g = 64
    src = edge_index_p[0]
    dst = edge_index_p[1]

    wp1 = _prep_wp(W1b, b1b, d, 32)
    wp2 = _prep_wp(W2b, b2b, 32, 16)
    zeros128 = jnp.zeros((n, 128), jnp.float32)

    ea_t = edge_attr_p.T

    bounds = [(0, 40960), (40960, 81920), (81920, 122880), (122880, 160000)]

    def conv(x, wa, ba, wp, out_c):
        parts = []
        for lo, hi in bounds:
            xj = _sc_gather(x, lax.slice(src, (lo,), (hi,)))
            m = _tc_msg(lax.slice(ea_t, (0, lo), (4, hi)), xj, wa, ba, wp,
                        out_c, eb=1280)
            p = _sc_scatter_add(m, lax.slice(dst, (lo,), (hi,)), n, zeros128)
            parts += [p[0], p[1]]
        return parts

    x1 = _tc_node(x_p, conv(x_p, W1a, b1a, wp1, 32), root1, bias1, nb=1000)
    parts2 = conv(x1, W2a, b2a, wp2, 16)
    return _tc_final(x1, parts2, x_p_batch.reshape(n, 1),
                     root2, bias2, lin1_w, lin1_b, lin2_w, lin2_b, g, nb=1000)


# speedup vs baseline: 3.1289x; 1.0766x over previous
import functools

import jax
import jax.numpy as jnp
from jax import lax
from jax.experimental import pallas as pl
from jax.experimental.pallas import tpu as pltpu
from jax.experimental.pallas import tpu_sc as plsc

_NC = 2
_NS = 16
_NW = _NC * _NS
_C = 128


def _sc_gather(table, idx):
    n_rows, d = table.shape
    e = idx.shape[0]
    per_w = e // _NW
    nchunk = (per_w + _C - 1) // _C
    last_off = per_w - _C
    mesh = plsc.VectorSubcoreMesh(core_axis_name="c", subcore_axis_name="s")

    @functools.partial(
        pl.kernel,
        out_type=jax.ShapeDtypeStruct((e, d), table.dtype),
        mesh=mesh,
        scratch_types=[
            pltpu.VMEM((_C,), jnp.int32),
            pltpu.VMEM((_C, d), table.dtype),
            pltpu.SemaphoreType.DMA,
        ],
    )
    def k(table_hbm, idx_hbm, out_hbm, idx_v, rows_v, sem):
        wid = lax.axis_index("s") * _NC + lax.axis_index("c")
        base = wid * per_w

        def body(j, carry):
            off = base + jnp.minimum(j * _C, last_off)
            pltpu.sync_copy(idx_hbm.at[pl.ds(off, _C)], idx_v)
            pltpu.async_copy(table_hbm.at[idx_v], rows_v, sem).wait()
            pltpu.sync_copy(rows_v, out_hbm.at[pl.ds(off, _C)])
            return carry

        lax.fori_loop(0, nchunk, body, 0, unroll=False)

    return k(table, idx)


def _sc_scatter_add(msgs, dsts, n_nodes, zeros_hbm):
    f = msgs[0].shape[1]
    plans = []
    for m in msgs:
        e = m.shape[0]
        per_w = e // _NW
        full = per_w // _C
        plans.append((per_w, full, per_w - full * _C))
    max_tail = max(p[2] for p in plans)
    rows_per_tile = 640
    mesh = plsc.VectorSubcoreMesh(core_axis_name="c", subcore_axis_name="s")

    scratch = [
        pltpu.VMEM((_C,), jnp.int32),
        pltpu.VMEM((_C, f), jnp.float32),
    ]
    if max_tail:
        scratch += [pltpu.VMEM((max_tail,), jnp.int32),
                    pltpu.VMEM((max_tail, f), jnp.float32)]
    scratch.append(pltpu.VMEM_SHARED((n_nodes, f), jnp.float32))

    @functools.partial(
        pl.kernel,
        out_type=jax.ShapeDtypeStruct((_NC, n_nodes, f), jnp.float32),
        mesh=mesh,
        scratch_types=scratch,
    )
    def k(*refs):
        nm = len(msgs)
        msg_hbms = refs[:nm]
        dst_hbms = refs[nm:2 * nm]
        zero_hbm = refs[2 * nm]
        out_hbm = refs[2 * nm + 1]
        idx_v, rows_v = refs[2 * nm + 2], refs[2 * nm + 3]
        if max_tail:
            idxt_v, rowst_v, acc_sh = refs[2 * nm + 4:]
        else:
            acc_sh, = refs[2 * nm + 4:]
        c = lax.axis_index("c")
        s = lax.axis_index("s")
        wid = s * _NC + c

        n_full_z = n_nodes // rows_per_tile
        rem_z = n_nodes - n_full_z * rows_per_tile

        @pl.when(s < n_full_z)
        def _():
            pltpu.sync_copy(zero_hbm.at[pl.ds(s * rows_per_tile, rows_per_tile)],
                            acc_sh.at[pl.ds(s * rows_per_tile, rows_per_tile)])

        if rem_z:
            @pl.when(s == n_full_z)
            def _():
                pltpu.sync_copy(zero_hbm.at[pl.ds(n_full_z * rows_per_tile, rem_z)],
                                acc_sh.at[pl.ds(n_full_z * rows_per_tile, rem_z)])

        plsc.subcore_barrier()

        for (per_w, full, tail), msg_hbm, dst_hbm in zip(plans, msg_hbms,
                                                         dst_hbms):
            base = wid * per_w

            def body(j, carry, msg_hbm=msg_hbm, dst_hbm=dst_hbm, base=base):
                off = base + j * _C
                pltpu.sync_copy(dst_hbm.at[pl.ds(off, _C)], idx_v)
                pltpu.sync_copy(msg_hbm.at[pl.ds(off, _C)], rows_v)
                pltpu.sync_copy(rows_v, acc_sh.at[idx_v], add=True)
                return carry

            lax.fori_loop(0, full, body, 0, unroll=False)

            if tail:
                assert tail == max_tail
                off = base + full * _C
                pltpu.sync_copy(dst_hbm.at[pl.ds(off, tail)], idxt_v)
                pltpu.sync_copy(msg_hbm.at[pl.ds(off, tail)], rowst_v)
                pltpu.sync_copy(rowst_v, acc_sh.at[idxt_v], add=True)

        plsc.subcore_barrier()

        @pl.when(s < n_full_z)
        def _():
            pltpu.sync_copy(acc_sh.at[pl.ds(s * rows_per_tile, rows_per_tile)],
                            out_hbm.at[c, pl.ds(s * rows_per_tile, rows_per_tile)])

        if rem_z:
            @pl.when(s == n_full_z)
            def _():
                pltpu.sync_copy(acc_sh.at[pl.ds(n_full_z * rows_per_tile, rem_z)],
                                out_hbm.at[c, pl.ds(n_full_z * rows_per_tile, rem_z)])

    return k(*msgs, *dsts, zeros_hbm)


def _msg_body(in_c, eat_ref, xj_ref, wrep_ref, brep_ref, wp_ref, f_ref,
              out_ref):
    dn = (((0,), (0,)), ((), ()))
    hx = jnp.maximum(
        lax.dot_general(eat_ref[...].astype(jnp.bfloat16), wrep_ref[...], dn,
                        preferred_element_type=jnp.float32) + brep_ref[...],
        0.0)
    xjb = xj_ref[...][:, :in_c].astype(jnp.bfloat16)
    t = jnp.dot(xjb, wp_ref[...], preferred_element_type=jnp.float32)
    msg = jnp.dot((hx * t).astype(jnp.bfloat16), f_ref[...],
                  preferred_element_type=jnp.float32)
    eb = msg.shape[0]
    out_ref[...] = jnp.concatenate(
        [msg, jnp.zeros((eb, 128 - msg.shape[1]), jnp.float32)], axis=1)


def _tc_msg(ea_t, xj, wa, ba, wp, out_c, eb):
    e, w_in = xj.shape
    in_c = wp.shape[0]
    ko = wp.shape[1]
    bf = jnp.bfloat16
    wrep = jnp.concatenate(
        [jnp.repeat(wa, out_c, axis=1), jnp.zeros((4, out_c), jnp.float32)],
        axis=1)
    brep = jnp.concatenate(
        [jnp.repeat(ba.reshape(1, 16), out_c, axis=1),
         jnp.ones((1, out_c), jnp.float32)], axis=1)
    fold = jnp.tile(jnp.eye(out_c, dtype=jnp.float32), (17, 1))
    grid = e // eb
    return pl.pallas_call(
        functools.partial(_msg_body, in_c),
        grid=(grid,),
        in_specs=[
            pl.BlockSpec((4, eb), lambda i: (0, i)),
            pl.BlockSpec((eb, w_in), lambda i: (i, 0)),
            pl.BlockSpec((4, ko), lambda i: (0, 0)),
            pl.BlockSpec((1, ko), lambda i: (0, 0)),
            pl.BlockSpec((in_c, ko), lambda i: (0, 0)),
            pl.BlockSpec((ko, out_c), lambda i: (0, 0)),
        ],
        out_specs=pl.BlockSpec((eb, 128), lambda i: (i, 0)),
        out_shape=jax.ShapeDtypeStruct((e, 128), jnp.float32),
    )(ea_t, xj, wrep.astype(bf), brep, wp.astype(bf), fold.astype(bf))


def _node_body(n_agg, x_ref, *refs):
    aggs = refs[:n_agg]
    root_ref, bias_ref, o_ref = refs[n_agg:]
    nb = x_ref.shape[0]
    out_c = root_ref.shape[1]
    v = jnp.dot(x_ref[...], root_ref[...], preferred_element_type=jnp.float32)
    for a in aggs:
        v = v + a[...][:, :out_c]
    v = jnp.maximum(v + bias_ref[...], 0.0)
    o_ref[...] = jnp.concatenate(
        [v, jnp.zeros((nb, 128 - out_c), jnp.float32)], axis=1)


def _tc_node(x, aggs, root, bias, nb):
    n, in_c = x.shape
    out_c = root.shape[1]
    return pl.pallas_call(
        functools.partial(_node_body, len(aggs)),
        grid=(n // nb,),
        in_specs=[pl.BlockSpec((nb, in_c), lambda i: (i, 0))]
        + [pl.BlockSpec((nb, a.shape[1]), lambda i: (i, 0)) for a in aggs]
        + [
            pl.BlockSpec((in_c, out_c), lambda i: (0, 0)),
            pl.BlockSpec((1, out_c), lambda i: (0, 0)),
        ],
        out_specs=pl.BlockSpec((nb, 128), lambda i: (i, 0)),
        out_shape=jax.ShapeDtypeStruct((n, 128), jnp.float32),
    )(x, *aggs, root, bias.reshape(1, out_c))


def _final_body(n_agg, x_ref, *refs):
    aggs = refs[:n_agg]
    (b_ref, root_ref, bias_ref, l1w_ref, l1b_ref, l2w_ref, l2b_ref,
     o_ref, sums, cnts) = refs[n_agg:]
    i = pl.program_id(0)
    nb = x_ref.shape[0]
    g = sums.shape[0]

    @pl.when(i == 0)
    def _():
        sums[...] = jnp.zeros_like(sums)
        cnts[...] = jnp.zeros_like(cnts)

    in_c = root_ref.shape[0]
    out_c = root_ref.shape[1]
    v = jnp.dot(x_ref[...][:, :in_c], root_ref[...],
                preferred_element_type=jnp.float32)
    for a in aggs:
        v = v + a[...][:, :out_c]
    x2 = jnp.maximum(v + bias_ref[...], 0.0)
    gio = lax.broadcasted_iota(jnp.int32, (nb, g), 1)
    mask = (gio == b_ref[...]).astype(jnp.float32)
    dn = (((0,), (0,)), ((), ()))
    sums[...] += lax.dot_general(mask, x2, dn,
                                 preferred_element_type=jnp.float32)
    cnts[...] += lax.dot_general(mask, jnp.ones((nb, 1), jnp.float32), dn,
                                 preferred_element_type=jnp.float32)

    @pl.when(i == pl.num_programs(0) - 1)
    def _():
        pooled = sums[...] / jnp.maximum(cnts[...], 1.0)
        o1 = jnp.dot(pooled, l1w_ref[...],
                     preferred_element_type=jnp.float32) + l1b_ref[...]
        o_ref[...] = jnp.dot(o1, l2w_ref[...],
                             preferred_element_type=jnp.float32) + l2b_ref[...]


def _tc_final(x1, aggs, batch, root, bias, l1w, l1b, l2w, l2b, g, nb):
    n, w_in = x1.shape
    in_c, out_c = root.shape
    return pl.pallas_call(
        functools.partial(_final_body, len(aggs)),
        grid=(n // nb,),
        in_specs=[pl.BlockSpec((nb, w_in), lambda i: (i, 0))]
        + [pl.BlockSpec((nb, a.shape[1]), lambda i: (i, 0)) for a in aggs]
        + [
            pl.BlockSpec((nb, 1), lambda i: (i, 0)),
            pl.BlockSpec((in_c, out_c), lambda i: (0, 0)),
            pl.BlockSpec((1, out_c), lambda i: (0, 0)),
            pl.BlockSpec((16, 8), lambda i: (0, 0)),
            pl.BlockSpec((1, 8), lambda i: (0, 0)),
            pl.BlockSpec((8, 1), lambda i: (0, 0)),
            pl.BlockSpec((1, 1), lambda i: (0, 0)),
        ],
        out_specs=pl.BlockSpec((g, 1), lambda i: (0, 0)),
        out_shape=jax.ShapeDtypeStruct((g, 1), jnp.float32),
        scratch_shapes=[
            pltpu.VMEM((g, out_c), jnp.float32),
            pltpu.VMEM((g, 1), jnp.float32),
        ],
    )(x1, *aggs, batch, root, bias.reshape(1, out_c),
      l1w, l1b.reshape(1, 8), l2w, l2b.reshape(1, 1))


def _prep_wp(wb, bb, in_c, out_c):
    w2r = wb.reshape(16, in_c, out_c)
    return jnp.concatenate(
        [jnp.transpose(w2r, (1, 0, 2)).reshape(in_c, 16 * out_c),
         bb.reshape(in_c, out_c)], axis=1)


def kernel(x_p, x_d, edge_attr_p, edge_attr_d, x_p_batch, edge_index_p,
           W1a, b1a, W1b, b1b, root1, bias1,
           W2a, b2a, W2b, b2b, root2, bias2,
           lin1_w, lin1_b, lin2_w, lin2_b):
    n, d = x_p.shape
    e = edge_index_p.shape[1]
    g = 64
    src = edge_index_p[0]
    dst = edge_index_p[1]

    wp1 = _prep_wp(W1b, b1b, d, 32)
    wp2 = _prep_wp(W2b, b2b, 32, 16)
    zeros128 = jnp.zeros((n, 128), jnp.float32)

    ea_t = edge_attr_p.T

    bounds = [(0, 40960), (40960, 81920), (81920, 122880), (122880, 160000)]

    def conv(x, wa, ba, wp, out_c, zeros):
        msgs = []
        for lo, hi in bounds:
            xj = _sc_gather(x, lax.slice(src, (lo,), (hi,)))
            msgs.append(_tc_msg(lax.slice(ea_t, (0, lo), (4, hi)), xj, wa, ba,
                                wp, out_c, eb=1280))
        parts = []
        for j in range(0, len(bounds), 2):
            sub = [lax.slice(dst, (lo,), (hi,)) for lo, hi in bounds[j:j + 2]]
            p = _sc_scatter_add(msgs[j:j + 2], sub, n, zeros)
            parts += [p[0], p[1]]
        return parts

    x1 = _tc_node(x_p, conv(x_p, W1a, b1a, wp1, 32, zeros128), root1, bias1,
                  nb=1000)
    parts2 = conv(x1, W2a, b2a, wp2, 16, zeros128)
    return _tc_final(x1, parts2, x_p_batch.reshape(n, 1),
                     root2, bias2, lin1_w, lin1_b, lin2_w, lin2_b, g, nb=1000)
